# Initial kernel scaffold; baseline (speedup 1.0000x reference)
import jax
import jax.numpy as jnp
from jax.experimental import pallas as pl


def kernel(h, A, W1_w, W1_b, a1, W2_w, W2_b, a2, W3_w, W3_b, FL_w, FL_b):
    raise NotImplementedError("write your pallas kernel here")


# speedup vs baseline: 65.3524x; 65.3524x over previous
import jax
import jax.numpy as jnp
from jax.experimental import pallas as pl
from jax.experimental.pallas import tpu as pltpu

_H = 4
_C = 128


def _attention(Wh, mask, a_ref):
    outs = []
    for hd in range(_H):
        Wh_h = Wh[:, hd * _C:(hd + 1) * _C]
        a_l = a_ref[hd:hd + 1, :_C]
        a_r = a_ref[hd:hd + 1, _C:]
        e = jax.lax.dot_general(Wh_h, a_l, (((1,), (1,)), ((), ())),
                                preferred_element_type=jnp.float32)
        f = jax.lax.dot_general(a_r, Wh_h, (((1,), (1,)), ((), ())),
                                preferred_element_type=jnp.float32)
        logits = e + f
        logits = jnp.where(logits >= 0, logits, 0.01 * logits)
        att = jnp.where(mask, logits, -jnp.inf)
        m = jnp.max(att, axis=1, keepdims=True)
        p = jnp.exp(att - m)
        s = jnp.sum(p, axis=1, keepdims=True)
        probs = p / s
        outs.append(jnp.dot(probs, Wh_h, preferred_element_type=jnp.float32))
    return jnp.concatenate(outs, axis=1)


def _gat_kernel(h_ref, A_ref, w1t_ref, b1_ref, a1_ref, w2t_ref, b2_ref,
                a2_ref, w3t_ref, b3_ref, flt_ref, flb_ref, h3_ref, out_ref):
    mask = A_ref[...] != 0
    Wh1 = jnp.dot(h_ref[...], w1t_ref[...],
                  preferred_element_type=jnp.float32) + b1_ref[...]
    h1 = _attention(Wh1, mask, a1_ref)
    h2 = jnp.dot(h1, w3t_ref[...],
                 preferred_element_type=jnp.float32) + b3_ref[...]
    Wh2 = jnp.dot(h2, w2t_ref[...],
                  preferred_element_type=jnp.float32) + b2_ref[...]
    h3 = _attention(Wh2, mask, a2_ref)
    h3_ref[...] = h3
    out_ref[...] = jnp.dot(h3, flt_ref[...],
                           preferred_element_type=jnp.float32) + flb_ref[...]


def kernel(h, A, W1_w, W1_b, a1, W2_w, W2_b, a2, W3_w, W3_b, FL_w, FL_b):
    b, n, _ = h.shape
    nc = FL_w.shape[0]
    h2d = h.reshape(n, -1)
    A2d = A.reshape(n, n)
    out_shapes = (
        jax.ShapeDtypeStruct((n, _H * _C), jnp.float32),
        jax.ShapeDtypeStruct((n, nc), jnp.float32),
    )
    h3, out = pl.pallas_call(
        _gat_kernel,
        out_shape=out_shapes,
    )(h2d, A2d, W1_w.T, W1_b.reshape(1, -1), a1, W2_w.T,
      W2_b.reshape(1, -1), a2, W3_w.T, W3_b.reshape(1, -1), FL_w.T,
      FL_b.reshape(1, -1))
    return (h3.reshape(b, n, _H * _C), out.reshape(b, n, nc))


# speedup vs baseline: 107.5874x; 1.6463x over previous
import jax
import jax.numpy as jnp
from jax.experimental import pallas as pl
from jax.experimental.pallas import tpu as pltpu

_H = 4
_C = 128

_DNT = (((1,), (1,)), ((), ()))


def _matmul_t(x, w):
    return jax.lax.dot_general(x, w, _DNT, preferred_element_type=jnp.float32)


def _attention(Wh, mask, a_ref):
    outs = []
    for hd in range(_H):
        Wh_h = Wh[:, hd * _C:(hd + 1) * _C]
        a_l = a_ref[hd:hd + 1, :_C]
        a_r = a_ref[hd:hd + 1, _C:]
        e = _matmul_t(Wh_h, a_l)
        f = _matmul_t(a_r, Wh_h)
        logits = e + f
        logits = jnp.where(logits >= 0, logits, 0.01 * logits)
        att = jnp.where(mask, logits, -jnp.inf)
        m = jnp.max(att, axis=1, keepdims=True)
        p = jnp.exp(att - m)
        s = jnp.sum(p, axis=1, keepdims=True)
        agg = jnp.dot(p, Wh_h, preferred_element_type=jnp.float32)
        outs.append(agg * (1.0 / s))
    return jnp.concatenate(outs, axis=1)


def _gat_kernel(h_ref, A_ref, w1_ref, b1_ref, a1_ref, w2_ref, b2_ref,
                a2_ref, w3_ref, b3_ref, fl_ref, flb_ref, h3_ref, out_ref):
    mask = A_ref[...] != 0
    Wh1 = _matmul_t(h_ref[...], w1_ref[...]) + b1_ref[...]
    h1 = _attention(Wh1, mask, a1_ref)
    h2 = _matmul_t(h1, w3_ref[...]) + b3_ref[...]
    Wh2 = _matmul_t(h2, w2_ref[...]) + b2_ref[...]
    h3 = _attention(Wh2, mask, a2_ref)
    h3_ref[...] = h3
    out_ref[...] = _matmul_t(h3, fl_ref[...]) + flb_ref[...]


def kernel(h, A, W1_w, W1_b, a1, W2_w, W2_b, a2, W3_w, W3_b, FL_w, FL_b):
    b, n, _ = h.shape
    nc = FL_w.shape[0]
    h2d = h.reshape(n, -1)
    A2d = A.reshape(n, n)
    out_shapes = (
        jax.ShapeDtypeStruct((n, _H * _C), jnp.float32),
        jax.ShapeDtypeStruct((n, nc), jnp.float32),
    )
    h3, out = pl.pallas_call(
        _gat_kernel,
        out_shape=out_shapes,
    )(h2d, A2d, W1_w, W1_b.reshape(1, -1), a1, W2_w,
      W2_b.reshape(1, -1), a2, W3_w, W3_b.reshape(1, -1), FL_w,
      FL_b.reshape(1, -1))
    return (h3.reshape(b, n, _H * _C), out.reshape(b, n, nc))
